# baseline (speedup 1.0000x reference)
import functools

import jax
import jax.numpy as jnp
from jax import lax
from jax.experimental import pallas as pl
from jax.experimental.pallas import tpu as pltpu
from jax.experimental.pallas import tpu_sc as plsc

_NC = 2
_NS = 16
_NW = _NC * _NS


def _wid():
    return lax.axis_index("s") * _NC + lax.axis_index("c")


def _gru_block(h_ref, v_ref, wz_ref, wr_ref, wq_ref, o_ref):
    h = h_ref[...]
    v = v_ref[...]
    Wz = wz_ref[...]
    Wr = wr_ref[...]
    Wq = wq_ref[...]
    D = h.shape[1]

    def mm(a, b):
        return jax.lax.dot_general(
            a, b, (((1,), (0,)), ((), ())), preferred_element_type=jnp.float32
        )

    z = jax.nn.sigmoid(mm(h, Wz[:D]) + mm(v, Wz[D:]))
    r = jax.nn.sigmoid(mm(h, Wr[:D]) + mm(v, Wr[D:]))
    q = jnp.tanh(mm(r * h, Wq[:D]) + mm(v, Wq[D:]))
    o_ref[...] = (1.0 - z) * h + z * q


@functools.cache
def _make_gather(M, B, D, W):
    CPW = B // _NW
    NWIN = CPW // W
    K = W // 128
    mesh = plsc.VectorSubcoreMesh(core_axis_name="c", subcore_axis_name="s")

    @functools.partial(
        pl.kernel,
        out_type=jax.ShapeDtypeStruct((B, D), jnp.float32),
        mesh=mesh,
        scratch_types=[
            pltpu.VMEM((K, 128), jnp.int32),
            pltpu.VMEM((W, D), jnp.float32),
            pltpu.SemaphoreType.DMA,
        ],
        compiler_params=pltpu.CompilerParams(use_tc_tiling_on_sc=False),
    )
    def gather(mem_hbm, idx_hbm, out_hbm, idx_v, rows_v, sem):
        w = _wid()
        rbase = w * (CPW // 128)

        @pl.loop(0, NWIN)
        def _win(g):
            r0 = rbase + g * K
            pltpu.sync_copy(idx_hbm.at[pl.ds(r0, K)], idx_v)
            cps = [
                pltpu.async_copy(
                    mem_hbm.at[idx_v.at[j]],
                    rows_v.at[pl.ds(j * 128, 128)],
                    sem,
                )
                for j in range(K)
            ]
            for cp in cps:
                cp.wait()
            pltpu.sync_copy(rows_v, out_hbm.at[pl.ds(r0 * 128, W)])

    return gather


@functools.cache
def _make_scatter(M, B, D, IW, CH):
    MW = M // _NW
    NWIN = B // IW
    NV = IW // 16
    LR = IW // CH
    mesh = plsc.VectorSubcoreMesh(core_axis_name="c", subcore_axis_name="s")

    @functools.partial(
        pl.kernel,
        out_type=(),
        mesh=mesh,
        scratch_types=[
            pltpu.VMEM((MW,), jnp.int32),
            pltpu.VMEM((IW,), jnp.int32),
            pltpu.VMEM((LR, CH), jnp.int32),
            pltpu.VMEM((LR, CH), jnp.int32),
            pltpu.VMEM((CH, D), jnp.float32),
            pltpu.VMEM((16, D), jnp.float32),
            pltpu.VMEM((16,), jnp.int32),
            pltpu.SemaphoreType.DMA,
        ],
        compiler_params=pltpu.CompilerParams(
            use_tc_tiling_on_sc=False, needs_layout_passes=False),
    )
    def scatter(mem_ref, idx_hbm, upd_hbm,
                winp, idxv, plist, rlist, rows, frows, fidx, sem):
        w = _wid()
        base = w * MW
        lane = lax.iota(jnp.int32, 16)

        fidx[...] = jnp.full((16,), base, jnp.int32)
        pltpu.async_copy(mem_ref.at[fidx], frows, sem).wait()
        winp[pl.ds(0, 16)] = jnp.full((16,), -1, jnp.int32)

        @pl.loop(0, NWIN)
        def _p1(g):
            pltpu.sync_copy(idx_hbm.at[pl.ds(g * IW, IW)], idxv)

            @pl.loop(0, NV)
            def _v(j):
                v = idxv[pl.ds(j * 16, 16)]
                a = v - base
                m = (a >= 0) & (a < MW)
                _, lastm = plsc.scan_count(v, m)
                posv = (g * IW + j * 16) + lane
                plsc.store_scatter(winp, [a], posv, mask=m & lastm)

        for r in range(LR):
            for c in range(CH // 16):
                plist[r, pl.ds(c * 16, 16)] = jnp.zeros((16,), jnp.int32)
                rlist[r, pl.ds(c * 16, 16)] = jnp.full((16,), base, jnp.int32)

        @pl.loop(0, NWIN)
        def _p2(g):
            pltpu.sync_copy(idx_hbm.at[pl.ds(g * IW, IW)], idxv)

            @pl.loop(0, NV, init_carry=jnp.zeros((16,), jnp.int32))
            def _v(j, noff):
                v = idxv[pl.ds(j * 16, 16)]
                a = v - base
                m = (a >= 0) & (a < MW)
                wp = plsc.load_gather(winp, [a], mask=m)
                posv = (g * IW + j * 16) + lane
                k = m & (wp == posv)
                ki = k.astype(jnp.int32)
                tgt = noff + plsc.cumsum(ki) - 1
                tr = lax.shift_right_logical(tgt, 7)
                tc = lax.bitwise_and(tgt, 127)
                plsc.store_scatter(plist, [tr, tc], posv, mask=k)
                plsc.store_scatter(rlist, [tr, tc], v, mask=k)
                return noff + plsc.all_reduce_population_count(k)

            n = lax.reduce_max(_v, (0,))
            nch = lax.shift_right_logical(n + (CH - 1), 7)

            @pl.loop(0, nch)
            def _f(i):
                pltpu.async_copy(upd_hbm.at[plist.at[i]], rows, sem).wait()
                pltpu.async_copy(rows, mem_ref.at[rlist.at[i]], sem).wait()

        pltpu.async_copy(frows, mem_ref.at[fidx], sem).wait()
        w0 = winp[pl.ds(0, 16)]
        val = lax.reduce_max(
            jnp.where(lane == 0, w0, jnp.full((16,), -2147483647, jnp.int32)),
            (0,))

        @pl.when(val >= 0)
        def _fix():
            fidx[...] = jnp.full((16,), val, jnp.int32)
            pltpu.async_copy(upd_hbm.at[fidx], frows, sem).wait()
            fidx[...] = jnp.full((16,), base, jnp.int32)
            pltpu.async_copy(frows, mem_ref.at[fidx], sem).wait()

    return scatter


def kernel(mem, val, Wz, Wr, Wq, idx):
    M, D = mem.shape
    B = idx.shape[0]
    idx = idx.astype(jnp.int32)

    h = _make_gather(M, B, D, 512)(mem, idx.reshape(B // 128, 128))

    BLK = 6912
    h_new = pl.pallas_call(
        _gru_block,
        grid=(B // BLK,),
        in_specs=[
            pl.BlockSpec((BLK, D), lambda i: (i, 0)),
            pl.BlockSpec((BLK, D), lambda i: (i, 0)),
            pl.BlockSpec((2 * D, D), lambda i: (0, 0)),
            pl.BlockSpec((2 * D, D), lambda i: (0, 0)),
            pl.BlockSpec((2 * D, D), lambda i: (0, 0)),
        ],
        out_specs=pl.BlockSpec((BLK, D), lambda i: (i, 0)),
        out_shape=jax.ShapeDtypeStruct((B, D), jnp.float32),
    )(h, val, Wz, Wr, Wq)

    mem_ref = jax.new_ref(mem)
    _make_scatter(M, B, D, 4096, 128)(mem_ref, idx, h_new)
    return mem_ref[...]


# speedup vs baseline: 1.0034x; 1.0034x over previous
import functools

import jax
import jax.numpy as jnp
from jax import lax
from jax.experimental import pallas as pl
from jax.experimental.pallas import tpu as pltpu
from jax.experimental.pallas import tpu_sc as plsc

_NC = 2
_NS = 16
_NW = _NC * _NS


def _wid():
    return lax.axis_index("s") * _NC + lax.axis_index("c")


def _gru_block(h_ref, v_ref, wz_ref, wr_ref, wq_ref, o_ref):
    h = h_ref[...]
    v = v_ref[...]
    Wz = wz_ref[...]
    Wr = wr_ref[...]
    Wq = wq_ref[...]
    D = h.shape[1]

    def mm(a, b):
        return jax.lax.dot_general(
            a, b, (((1,), (0,)), ((), ())), preferred_element_type=jnp.float32
        )

    z = jax.nn.sigmoid(mm(h, Wz[:D]) + mm(v, Wz[D:]))
    r = jax.nn.sigmoid(mm(h, Wr[:D]) + mm(v, Wr[D:]))
    q = jnp.tanh(mm(r * h, Wq[:D]) + mm(v, Wq[D:]))
    o_ref[...] = (1.0 - z) * h + z * q


@functools.cache
def _make_gather(M, B, D, W):
    CPW = B // _NW
    NWIN = CPW // W
    K = W // 128
    mesh = plsc.VectorSubcoreMesh(core_axis_name="c", subcore_axis_name="s")

    @functools.partial(
        pl.kernel,
        out_type=jax.ShapeDtypeStruct((B, D), jnp.float32),
        mesh=mesh,
        scratch_types=[
            pltpu.VMEM((K, 128), jnp.int32),
            pltpu.VMEM((W, D), jnp.float32),
            pltpu.SemaphoreType.DMA,
        ],
        compiler_params=pltpu.CompilerParams(use_tc_tiling_on_sc=False),
    )
    def gather(mem_hbm, idx_hbm, out_hbm, idx_v, rows_v, sem):
        w = _wid()
        rbase = w * (CPW // 128)

        @pl.loop(0, NWIN)
        def _win(g):
            r0 = rbase + g * K
            pltpu.sync_copy(idx_hbm.at[pl.ds(r0, K)], idx_v)
            cps = [
                pltpu.async_copy(
                    mem_hbm.at[idx_v.at[j]],
                    rows_v.at[pl.ds(j * 128, 128)],
                    sem,
                )
                for j in range(K)
            ]
            for cp in cps:
                cp.wait()
            pltpu.sync_copy(rows_v, out_hbm.at[pl.ds(r0 * 128, W)])

    return gather


@functools.cache
def _make_scatter(M, B, D, IW, CH):
    MW = M // _NW
    NWIN = B // IW
    NV = IW // 16
    LR = IW // CH
    mesh = plsc.VectorSubcoreMesh(core_axis_name="c", subcore_axis_name="s")

    @functools.partial(
        pl.kernel,
        out_type=(),
        mesh=mesh,
        scratch_types=[
            pltpu.VMEM((MW,), jnp.int32),
            pltpu.VMEM((IW,), jnp.int32),
            pltpu.VMEM((LR, CH), jnp.int32),
            pltpu.VMEM((LR, CH), jnp.int32),
            pltpu.VMEM((CH, D), jnp.float32),
            pltpu.VMEM((16, D), jnp.float32),
            pltpu.VMEM((16,), jnp.int32),
            pltpu.SemaphoreType.DMA,
        ],
        compiler_params=pltpu.CompilerParams(
            use_tc_tiling_on_sc=False, needs_layout_passes=False),
    )
    def scatter(mem_ref, idx_hbm, upd_hbm,
                winp, idxv, plist, rlist, rows, frows, fidx, sem):
        w = _wid()
        base = w * MW
        lane = lax.iota(jnp.int32, 16)

        fidx[...] = jnp.full((16,), base, jnp.int32)
        pltpu.async_copy(mem_ref.at[fidx], frows, sem).wait()
        winp[pl.ds(0, 16)] = jnp.full((16,), -1, jnp.int32)

        @pl.loop(0, NWIN)
        def _p1(g):
            pltpu.sync_copy(idx_hbm.at[pl.ds(g * IW, IW)], idxv)

            @pl.loop(0, NV, unroll=8)
            def _v(j):
                v = idxv[pl.ds(j * 16, 16)]
                a = v - base
                m = (a >= 0) & (a < MW)
                _, lastm = plsc.scan_count(v, m)
                posv = (g * IW + j * 16) + lane
                plsc.store_scatter(winp, [a], posv, mask=m & lastm)

        for r in range(LR):
            for c in range(CH // 16):
                plist[r, pl.ds(c * 16, 16)] = jnp.zeros((16,), jnp.int32)
                rlist[r, pl.ds(c * 16, 16)] = jnp.full((16,), base, jnp.int32)

        @pl.loop(0, NWIN)
        def _p2(g):
            pltpu.sync_copy(idx_hbm.at[pl.ds(g * IW, IW)], idxv)

            @pl.loop(0, NV, init_carry=jnp.zeros((16,), jnp.int32),
                     unroll=8)
            def _v(j, noff):
                v = idxv[pl.ds(j * 16, 16)]
                a = v - base
                m = (a >= 0) & (a < MW)
                wp = plsc.load_gather(winp, [a], mask=m)
                posv = (g * IW + j * 16) + lane
                k = m & (wp == posv)
                ki = k.astype(jnp.int32)
                tgt = noff + plsc.cumsum(ki) - 1
                tr = lax.shift_right_logical(tgt, 7)
                tc = lax.bitwise_and(tgt, 127)
                plsc.store_scatter(plist, [tr, tc], posv, mask=k)
                plsc.store_scatter(rlist, [tr, tc], v, mask=k)
                return noff + plsc.all_reduce_population_count(k)

            n = lax.reduce_max(_v, (0,))
            nch = lax.shift_right_logical(n + (CH - 1), 7)

            @pl.loop(0, nch)
            def _f(i):
                pltpu.async_copy(upd_hbm.at[plist.at[i]], rows, sem).wait()
                pltpu.async_copy(rows, mem_ref.at[rlist.at[i]], sem).wait()

        pltpu.async_copy(frows, mem_ref.at[fidx], sem).wait()
        w0 = winp[pl.ds(0, 16)]
        val = lax.reduce_max(
            jnp.where(lane == 0, w0, jnp.full((16,), -2147483647, jnp.int32)),
            (0,))

        @pl.when(val >= 0)
        def _fix():
            fidx[...] = jnp.full((16,), val, jnp.int32)
            pltpu.async_copy(upd_hbm.at[fidx], frows, sem).wait()
            fidx[...] = jnp.full((16,), base, jnp.int32)
            pltpu.async_copy(frows, mem_ref.at[fidx], sem).wait()

    return scatter


def kernel(mem, val, Wz, Wr, Wq, idx):
    M, D = mem.shape
    B = idx.shape[0]
    idx = idx.astype(jnp.int32)

    h = _make_gather(M, B, D, 512)(mem, idx.reshape(B // 128, 128))

    BLK = 6912
    h_new = pl.pallas_call(
        _gru_block,
        grid=(B // BLK,),
        in_specs=[
            pl.BlockSpec((BLK, D), lambda i: (i, 0)),
            pl.BlockSpec((BLK, D), lambda i: (i, 0)),
            pl.BlockSpec((2 * D, D), lambda i: (0, 0)),
            pl.BlockSpec((2 * D, D), lambda i: (0, 0)),
            pl.BlockSpec((2 * D, D), lambda i: (0, 0)),
        ],
        out_specs=pl.BlockSpec((BLK, D), lambda i: (i, 0)),
        out_shape=jax.ShapeDtypeStruct((B, D), jnp.float32),
    )(h, val, Wz, Wr, Wq)

    mem_ref = jax.new_ref(mem)
    _make_scatter(M, B, D, 4096, 128)(mem_ref, idx, h_new)
    return mem_ref[...]


# speedup vs baseline: 1.0506x; 1.0471x over previous
import functools

import jax
import jax.numpy as jnp
from jax import lax
from jax.experimental import pallas as pl
from jax.experimental.pallas import tpu as pltpu
from jax.experimental.pallas import tpu_sc as plsc

_NC = 2
_NS = 16
_NW = _NC * _NS


def _wid():
    return lax.axis_index("s") * _NC + lax.axis_index("c")


def _gru_block(h_ref, v_ref, wz_ref, wr_ref, wq_ref, o_ref):
    h = h_ref[...]
    v = v_ref[...]
    Wz = wz_ref[...]
    Wr = wr_ref[...]
    Wq = wq_ref[...]
    D = h.shape[1]

    def mm(a, b):
        return jax.lax.dot_general(
            a, b, (((1,), (0,)), ((), ())), preferred_element_type=jnp.float32
        )

    z = jax.nn.sigmoid(mm(h, Wz[:D]) + mm(v, Wz[D:]))
    r = jax.nn.sigmoid(mm(h, Wr[:D]) + mm(v, Wr[D:]))
    q = jnp.tanh(mm(r * h, Wq[:D]) + mm(v, Wq[D:]))
    o_ref[...] = (1.0 - z) * h + z * q


@functools.cache
def _make_gather(M, B, D, W):
    CPW = B // _NW
    NWIN = CPW // W
    K = W // 128
    mesh = plsc.VectorSubcoreMesh(core_axis_name="c", subcore_axis_name="s")

    @functools.partial(
        pl.kernel,
        out_type=jax.ShapeDtypeStruct((B, D), jnp.float32),
        mesh=mesh,
        scratch_types=[
            pltpu.VMEM((K, 128), jnp.int32),
            pltpu.VMEM((W, D), jnp.float32),
            pltpu.SemaphoreType.DMA,
        ],
        compiler_params=pltpu.CompilerParams(use_tc_tiling_on_sc=False),
    )
    def gather(mem_hbm, idx_hbm, out_hbm, idx_v, rows_v, sem):
        w = _wid()
        rbase = w * (CPW // 128)

        @pl.loop(0, NWIN)
        def _win(g):
            r0 = rbase + g * K
            pltpu.sync_copy(idx_hbm.at[pl.ds(r0, K)], idx_v)
            cps = [
                pltpu.async_copy(
                    mem_hbm.at[idx_v.at[j]],
                    rows_v.at[pl.ds(j * 128, 128)],
                    sem,
                )
                for j in range(K)
            ]
            for cp in cps:
                cp.wait()
            pltpu.sync_copy(rows_v, out_hbm.at[pl.ds(r0 * 128, W)])

    return gather


@functools.cache
def _make_scatter(M, B, D, IW, CH):
    MW = M // _NW
    NWIN = B // IW
    NV = IW // 16
    LR = IW // CH
    mesh = plsc.VectorSubcoreMesh(core_axis_name="c", subcore_axis_name="s")

    @functools.partial(
        pl.kernel,
        out_type=(),
        mesh=mesh,
        scratch_types=[
            pltpu.VMEM((MW,), jnp.int32),
            pltpu.VMEM((IW,), jnp.int32),
            pltpu.VMEM((IW,), jnp.int32),
            pltpu.VMEM((LR, CH), jnp.int32),
            pltpu.VMEM((LR, CH), jnp.int32),
            pltpu.VMEM((CH, D), jnp.float32),
            pltpu.VMEM((16, D), jnp.float32),
            pltpu.VMEM((16,), jnp.int32),
            pltpu.SemaphoreType.DMA,
            pltpu.SemaphoreType.DMA,
            pltpu.SemaphoreType.DMA,
        ],
        compiler_params=pltpu.CompilerParams(
            use_tc_tiling_on_sc=False, needs_layout_passes=False),
    )
    def scatter(mem_ref, idx_hbm, upd_hbm,
                winp, idxv0, idxv1, plist, rlist, rows, frows, fidx, sem, sem0, sem1):
        w = _wid()
        base = w * MW
        lane = lax.iota(jnp.int32, 16)

        fidx[...] = jnp.full((16,), base, jnp.int32)
        pltpu.async_copy(mem_ref.at[fidx], frows, sem).wait()
        winp[pl.ds(0, 16)] = jnp.full((16,), -1, jnp.int32)

        def _p1_body(g, buf):
            @pl.loop(0, NV, unroll=8)
            def _v(j):
                v = buf[pl.ds(j * 16, 16)]
                a = v - base
                m = (a >= 0) & (a < MW)
                _, lastm = plsc.scan_count(v, m)
                posv = (g * IW + j * 16) + lane
                plsc.store_scatter(winp, [a], posv, mask=m & lastm)

        def _load(g, buf, s):
            gw = lax.rem(g, NWIN)
            pltpu.async_copy(idx_hbm.at[pl.ds(gw * IW, IW)], buf, s)

        def _drain(buf, s):
            pltpu.make_async_copy(idx_hbm.at[pl.ds(0, IW)], buf, s).wait()

        _load(0, idxv0, sem0)

        @pl.loop(0, NWIN // 2)
        def _p1(t):
            g0 = 2 * t
            _drain(idxv0, sem0)
            _load(g0 + 1, idxv1, sem1)
            _p1_body(g0, idxv0)
            _drain(idxv1, sem1)
            _load(g0 + 2, idxv0, sem0)
            _p1_body(g0 + 1, idxv1)

        _drain(idxv0, sem0)

        for r in range(LR):
            for c in range(CH // 16):
                plist[r, pl.ds(c * 16, 16)] = jnp.zeros((16,), jnp.int32)
                rlist[r, pl.ds(c * 16, 16)] = jnp.full((16,), base, jnp.int32)

        def _p2_body(g, buf):
            @pl.loop(0, NV, init_carry=jnp.zeros((16,), jnp.int32),
                     unroll=8)
            def _v(j, noff):
                v = buf[pl.ds(j * 16, 16)]
                a = v - base
                m = (a >= 0) & (a < MW)
                wp = plsc.load_gather(winp, [a], mask=m)
                posv = (g * IW + j * 16) + lane
                k = m & (wp == posv)
                ki = k.astype(jnp.int32)
                tgt = noff + plsc.cumsum(ki) - 1
                tr = lax.shift_right_logical(tgt, 7)
                tc = lax.bitwise_and(tgt, 127)
                plsc.store_scatter(plist, [tr, tc], posv, mask=k)
                plsc.store_scatter(rlist, [tr, tc], v, mask=k)
                return noff + plsc.all_reduce_population_count(k)

            n = lax.reduce_max(_v, (0,))
            nch = lax.shift_right_logical(n + (CH - 1), 7)

            @pl.loop(0, nch)
            def _f(i):
                pltpu.async_copy(upd_hbm.at[plist.at[i]], rows, sem).wait()
                pltpu.async_copy(rows, mem_ref.at[rlist.at[i]], sem).wait()

        _load(0, idxv0, sem0)

        @pl.loop(0, NWIN // 2)
        def _p2(t):
            g0 = 2 * t
            _drain(idxv0, sem0)
            _load(g0 + 1, idxv1, sem1)
            _p2_body(g0, idxv0)
            _drain(idxv1, sem1)
            _load(g0 + 2, idxv0, sem0)
            _p2_body(g0 + 1, idxv1)

        _drain(idxv0, sem0)

        pltpu.async_copy(frows, mem_ref.at[fidx], sem).wait()
        w0 = winp[pl.ds(0, 16)]
        val = lax.reduce_max(
            jnp.where(lane == 0, w0, jnp.full((16,), -2147483647, jnp.int32)),
            (0,))

        @pl.when(val >= 0)
        def _fix():
            fidx[...] = jnp.full((16,), val, jnp.int32)
            pltpu.async_copy(upd_hbm.at[fidx], frows, sem).wait()
            fidx[...] = jnp.full((16,), base, jnp.int32)
            pltpu.async_copy(frows, mem_ref.at[fidx], sem).wait()

    return scatter


def kernel(mem, val, Wz, Wr, Wq, idx):
    M, D = mem.shape
    B = idx.shape[0]
    idx = idx.astype(jnp.int32)

    h = _make_gather(M, B, D, 512)(mem, idx.reshape(B // 128, 128))

    BLK = 6912
    h_new = pl.pallas_call(
        _gru_block,
        grid=(B // BLK,),
        in_specs=[
            pl.BlockSpec((BLK, D), lambda i: (i, 0)),
            pl.BlockSpec((BLK, D), lambda i: (i, 0)),
            pl.BlockSpec((2 * D, D), lambda i: (0, 0)),
            pl.BlockSpec((2 * D, D), lambda i: (0, 0)),
            pl.BlockSpec((2 * D, D), lambda i: (0, 0)),
        ],
        out_specs=pl.BlockSpec((BLK, D), lambda i: (i, 0)),
        out_shape=jax.ShapeDtypeStruct((B, D), jnp.float32),
    )(h, val, Wz, Wr, Wq)

    mem_ref = jax.new_ref(mem)
    _make_scatter(M, B, D, 4096, 128)(mem_ref, idx, h_new)
    return mem_ref[...]


# speedup vs baseline: 1.0573x; 1.0063x over previous
import functools

import jax
import jax.numpy as jnp
from jax import lax
from jax.experimental import pallas as pl
from jax.experimental.pallas import tpu as pltpu
from jax.experimental.pallas import tpu_sc as plsc

_NC = 2
_NS = 16
_NW = _NC * _NS


def _wid():
    return lax.axis_index("s") * _NC + lax.axis_index("c")


def _gru_block(h_ref, v_ref, wz_ref, wr_ref, wq_ref, o_ref):
    h = h_ref[...]
    v = v_ref[...]
    Wz = wz_ref[...]
    Wr = wr_ref[...]
    Wq = wq_ref[...]
    D = h.shape[1]

    def mm(a, b):
        return jax.lax.dot_general(
            a, b, (((1,), (0,)), ((), ())), preferred_element_type=jnp.float32
        )

    z = jax.nn.sigmoid(mm(h, Wz[:D]) + mm(v, Wz[D:]))
    r = jax.nn.sigmoid(mm(h, Wr[:D]) + mm(v, Wr[D:]))
    q = jnp.tanh(mm(r * h, Wq[:D]) + mm(v, Wq[D:]))
    o_ref[...] = (1.0 - z) * h + z * q


@functools.cache
def _make_gather(M, B, D, W):
    CPW = B // _NW
    NWIN = CPW // W
    K = W // 128
    mesh = plsc.VectorSubcoreMesh(core_axis_name="c", subcore_axis_name="s")

    @functools.partial(
        pl.kernel,
        out_type=jax.ShapeDtypeStruct((B, D), jnp.float32),
        mesh=mesh,
        scratch_types=[
            pltpu.VMEM((K, 128), jnp.int32),
            pltpu.VMEM((W, D), jnp.float32),
            pltpu.SemaphoreType.DMA,
        ],
        compiler_params=pltpu.CompilerParams(use_tc_tiling_on_sc=False),
    )
    def gather(mem_hbm, idx_hbm, out_hbm, idx_v, rows_v, sem):
        w = _wid()
        rbase = w * (CPW // 128)

        @pl.loop(0, NWIN)
        def _win(g):
            r0 = rbase + g * K
            pltpu.sync_copy(idx_hbm.at[pl.ds(r0, K)], idx_v)
            cps = [
                pltpu.async_copy(
                    mem_hbm.at[idx_v.at[j]],
                    rows_v.at[pl.ds(j * 128, 128)],
                    sem,
                )
                for j in range(K)
            ]
            for cp in cps:
                cp.wait()
            pltpu.sync_copy(rows_v, out_hbm.at[pl.ds(r0 * 128, W)])

    return gather


@functools.cache
def _make_scatter(M, B, D, IW, CH):
    MW = M // _NW
    NWIN = B // IW
    NV = IW // 16
    LR = IW // CH
    mesh = plsc.VectorSubcoreMesh(core_axis_name="c", subcore_axis_name="s")

    @functools.partial(
        pl.kernel,
        out_type=(),
        mesh=mesh,
        scratch_types=[
            pltpu.VMEM((MW,), jnp.int32),
            pltpu.VMEM((IW,), jnp.int32),
            pltpu.VMEM((IW,), jnp.int32),
            pltpu.VMEM((LR, CH), jnp.int32),
            pltpu.VMEM((LR, CH), jnp.int32),
            pltpu.VMEM((CH, D), jnp.float32),
            pltpu.VMEM((16, D), jnp.float32),
            pltpu.VMEM((16,), jnp.int32),
            pltpu.SemaphoreType.DMA,
            pltpu.SemaphoreType.DMA,
            pltpu.SemaphoreType.DMA,
        ],
        compiler_params=pltpu.CompilerParams(
            use_tc_tiling_on_sc=False, needs_layout_passes=False),
    )
    def scatter(mem_ref, idx_hbm, upd_hbm,
                winp, idxv0, idxv1, plist, rlist, rows, frows, fidx, sem, sem0, sem1):
        w = _wid()
        base = w * MW
        lane = lax.iota(jnp.int32, 16)

        fidx[...] = jnp.full((16,), base, jnp.int32)
        pltpu.async_copy(mem_ref.at[fidx], frows, sem).wait()
        winp[pl.ds(0, 16)] = jnp.full((16,), -1, jnp.int32)

        def _p1_body(g, buf):
            @pl.loop(0, NV, unroll=8)
            def _v(j):
                v = buf[pl.ds(j * 16, 16)]
                a = v - base
                m = (a >= 0) & (a < MW)
                _, lastm = plsc.scan_count(v, m)
                posv = (g * IW + j * 16) + lane
                plsc.store_scatter(winp, [a], posv, mask=m & lastm)

        def _load(g, buf, s):
            gw = lax.rem(g, NWIN)
            pltpu.async_copy(idx_hbm.at[pl.ds(gw * IW, IW)], buf, s)

        def _drain(buf, s):
            pltpu.make_async_copy(idx_hbm.at[pl.ds(0, IW)], buf, s).wait()

        _load(0, idxv0, sem0)

        @pl.loop(0, NWIN // 2)
        def _p1(t):
            g0 = 2 * t
            _drain(idxv0, sem0)
            _load(g0 + 1, idxv1, sem1)
            _p1_body(g0, idxv0)
            _drain(idxv1, sem1)
            _load(g0 + 2, idxv0, sem0)
            _p1_body(g0 + 1, idxv1)

        _drain(idxv0, sem0)

        for r in range(LR):
            for c in range(CH // 16):
                plist[r, pl.ds(c * 16, 16)] = jnp.zeros((16,), jnp.int32)
                rlist[r, pl.ds(c * 16, 16)] = jnp.full((16,), base, jnp.int32)

        def _p2_body(g, buf):
            @pl.loop(0, NV, init_carry=jnp.zeros((16,), jnp.int32),
                     unroll=8)
            def _v(j, noff):
                v = buf[pl.ds(j * 16, 16)]
                a = v - base
                m = (a >= 0) & (a < MW)
                wp = plsc.load_gather(winp, [a], mask=m)
                posv = (g * IW + j * 16) + lane
                k = m & (wp == posv)
                ki = k.astype(jnp.int32)
                tgt = noff + plsc.cumsum(ki) - 1
                tr = lax.shift_right_logical(tgt, 7)
                tc = lax.bitwise_and(tgt, 127)
                plsc.store_scatter(plist, [tr, tc], posv, mask=k)
                plsc.store_scatter(rlist, [tr, tc], v, mask=k)
                return noff + plsc.all_reduce_population_count(k)

            n = lax.reduce_max(_v, (0,))
            nch = lax.shift_right_logical(n + (CH - 1), 7)

            @pl.loop(0, nch)
            def _f(i):
                pltpu.async_copy(upd_hbm.at[plist.at[i]], rows, sem).wait()
                pltpu.async_copy(rows, mem_ref.at[rlist.at[i]], sem).wait()

        _load(0, idxv0, sem0)

        @pl.loop(0, NWIN // 2)
        def _p2(t):
            g0 = 2 * t
            _drain(idxv0, sem0)
            _load(g0 + 1, idxv1, sem1)
            _p2_body(g0, idxv0)
            _drain(idxv1, sem1)
            _load(g0 + 2, idxv0, sem0)
            _p2_body(g0 + 1, idxv1)

        _drain(idxv0, sem0)

        pltpu.async_copy(frows, mem_ref.at[fidx], sem).wait()
        w0 = winp[pl.ds(0, 16)]
        val = lax.reduce_max(
            jnp.where(lane == 0, w0, jnp.full((16,), -2147483647, jnp.int32)),
            (0,))

        @pl.when(val >= 0)
        def _fix():
            fidx[...] = jnp.full((16,), val, jnp.int32)
            pltpu.async_copy(upd_hbm.at[fidx], frows, sem).wait()
            fidx[...] = jnp.full((16,), base, jnp.int32)
            pltpu.async_copy(frows, mem_ref.at[fidx], sem).wait()

    return scatter


def kernel(mem, val, Wz, Wr, Wq, idx):
    M, D = mem.shape
    B = idx.shape[0]
    idx = idx.astype(jnp.int32)

    h = _make_gather(M, B, D, 512)(mem, idx.reshape(B // 128, 128))

    BLK = 6912
    h_new = pl.pallas_call(
        _gru_block,
        grid=(B // BLK,),
        in_specs=[
            pl.BlockSpec((BLK, D), lambda i: (i, 0)),
            pl.BlockSpec((BLK, D), lambda i: (i, 0)),
            pl.BlockSpec((2 * D, D), lambda i: (0, 0)),
            pl.BlockSpec((2 * D, D), lambda i: (0, 0)),
            pl.BlockSpec((2 * D, D), lambda i: (0, 0)),
        ],
        out_specs=pl.BlockSpec((BLK, D), lambda i: (i, 0)),
        out_shape=jax.ShapeDtypeStruct((B, D), jnp.float32),
    )(h, val, Wz, Wr, Wq)

    mem_ref = jax.new_ref(mem)
    _make_scatter(M, B, D, 8192, 128)(mem_ref, idx, h_new)
    return mem_ref[...]


# speedup vs baseline: 1.3169x; 1.2456x over previous
import functools

import jax
import jax.numpy as jnp
from jax import lax
from jax.experimental import pallas as pl
from jax.experimental.pallas import tpu as pltpu
from jax.experimental.pallas import tpu_sc as plsc

_NC = 2
_NS = 16
_NW = _NC * _NS


def _wid():
    return lax.axis_index("s") * _NC + lax.axis_index("c")


def _gru_block(h_ref, v_ref, wz_ref, wr_ref, wq_ref, o_ref):
    h = h_ref[...]
    v = v_ref[...]
    Wz = wz_ref[...]
    Wr = wr_ref[...]
    Wq = wq_ref[...]
    D = h.shape[1]

    def mm(a, b):
        return jax.lax.dot_general(
            a, b, (((1,), (0,)), ((), ())), preferred_element_type=jnp.float32
        )

    z = jax.nn.sigmoid(mm(h, Wz[:D]) + mm(v, Wz[D:]))
    r = jax.nn.sigmoid(mm(h, Wr[:D]) + mm(v, Wr[D:]))
    q = jnp.tanh(mm(r * h, Wq[:D]) + mm(v, Wq[D:]))
    o_ref[...] = (1.0 - z) * h + z * q


@functools.cache
def _make_gather(M, B, D, W):
    CPW = B // _NW
    NWIN = CPW // W
    K = W // 128
    mesh = plsc.VectorSubcoreMesh(core_axis_name="c", subcore_axis_name="s")

    @functools.partial(
        pl.kernel,
        out_type=jax.ShapeDtypeStruct((B, D), jnp.float32),
        mesh=mesh,
        scratch_types=[
            pltpu.VMEM((K, 128), jnp.int32),
            pltpu.VMEM((W, D), jnp.float32),
            pltpu.SemaphoreType.DMA,
        ],
        compiler_params=pltpu.CompilerParams(use_tc_tiling_on_sc=False),
    )
    def gather(mem_hbm, idx_hbm, out_hbm, idx_v, rows_v, sem):
        w = _wid()
        rbase = w * (CPW // 128)

        @pl.loop(0, NWIN)
        def _win(g):
            r0 = rbase + g * K
            pltpu.sync_copy(idx_hbm.at[pl.ds(r0, K)], idx_v)
            cps = [
                pltpu.async_copy(
                    mem_hbm.at[idx_v.at[j]],
                    rows_v.at[pl.ds(j * 128, 128)],
                    sem,
                )
                for j in range(K)
            ]
            for cp in cps:
                cp.wait()
            pltpu.sync_copy(rows_v, out_hbm.at[pl.ds(r0 * 128, W)])

    return gather


@functools.cache
def _make_scatter(M, B, D, IW, CH):
    MW = M // _NW
    NWIN = B // IW
    NV = IW // 16
    LR = IW // CH
    mesh = plsc.VectorSubcoreMesh(core_axis_name="c", subcore_axis_name="s")

    @functools.partial(
        pl.kernel,
        out_type=(),
        mesh=mesh,
        scratch_types=[
            pltpu.VMEM((MW,), jnp.int32),
            pltpu.VMEM((IW,), jnp.int32),
            pltpu.VMEM((IW,), jnp.int32),
            pltpu.VMEM((LR, CH), jnp.int32),
            pltpu.VMEM((LR, CH), jnp.int32),
            pltpu.VMEM((CH, D), jnp.float32),
            pltpu.VMEM((16, D), jnp.float32),
            pltpu.VMEM((16,), jnp.int32),
            pltpu.SemaphoreType.DMA,
            pltpu.SemaphoreType.DMA,
            pltpu.SemaphoreType.DMA,
        ],
        compiler_params=pltpu.CompilerParams(
            use_tc_tiling_on_sc=False, needs_layout_passes=False),
    )
    def scatter(mem_ref, idx_hbm, upd_hbm,
                winp, idxv0, idxv1, plist, rlist, rows, frows, fidx, sem, sem0, sem1):
        w = _wid()
        base = w * MW
        lane = lax.iota(jnp.int32, 16)

        fidx[...] = jnp.full((16,), base, jnp.int32)
        pltpu.async_copy(mem_ref.at[fidx], frows, sem).wait()
        @pl.loop(0, MW // 16, unroll=8)
        def _init(i):
            winp[pl.ds(i * 16, 16)] = jnp.full((16,), -1, jnp.int32)

        def _p1_body(g, buf):
            @pl.loop(0, NV, unroll=8)
            def _v(j):
                v = buf[pl.ds(j * 16, 16)]
                a = v - base
                m = (a >= 0) & (a < MW)
                _, lastm = plsc.scan_count(v, m)
                posv = (g * IW + j * 16) + lane
                plsc.store_scatter(winp, [a], posv, mask=m & lastm)

        def _load(g, buf, s):
            gw = lax.rem(g, NWIN)
            pltpu.async_copy(idx_hbm.at[pl.ds(gw * IW, IW)], buf, s)

        def _drain(buf, s):
            pltpu.make_async_copy(idx_hbm.at[pl.ds(0, IW)], buf, s).wait()

        _load(0, idxv0, sem0)

        @pl.loop(0, NWIN // 2)
        def _p1(t):
            g0 = 2 * t
            _drain(idxv0, sem0)
            _load(g0 + 1, idxv1, sem1)
            _p1_body(g0, idxv0)
            _drain(idxv1, sem1)
            _load(g0 + 2, idxv0, sem0)
            _p1_body(g0 + 1, idxv1)

        _drain(idxv0, sem0)

        for r in range(LR):
            for c in range(CH // 16):
                plist[r, pl.ds(c * 16, 16)] = jnp.zeros((16,), jnp.int32)
                rlist[r, pl.ds(c * 16, 16)] = jnp.full((16,), base, jnp.int32)

        BK = 2048
        zero16 = jnp.zeros((16,), jnp.int32)

        @pl.loop(0, MW // BK, init_carry=(zero16, zero16))
        def _p2(bk, carry):
            noff, done = carry

            @pl.loop(0, BK // 16, unroll=8, init_carry=noff)
            def _vv(i, nf):
                off = bk * BK + i * 16
                wp = winp[pl.ds(off, 16)]
                k = wp >= 0
                tgt = nf + plsc.cumsum(k.astype(jnp.int32)) - 1
                tr = lax.bitwise_and(lax.shift_right_logical(tgt, 7), LR - 1)
                tc = lax.bitwise_and(tgt, 127)
                plsc.store_scatter(plist, [tr, tc], wp, mask=k)
                plsc.store_scatter(rlist, [tr, tc], (base + off) + lane,
                                   mask=k)
                return nf + plsc.all_reduce_population_count(k)

            d0 = lax.reduce_max(done, (0,))
            d1 = lax.reduce_max(lax.shift_right_logical(_vv, 7), (0,))

            @pl.loop(d0, d1)
            def _f(c):
                cr = lax.bitwise_and(c, LR - 1)
                pltpu.async_copy(upd_hbm.at[plist.at[cr]], rows, sem).wait()
                pltpu.async_copy(rows, mem_ref.at[rlist.at[cr]], sem).wait()

            return _vv, jnp.full((16,), d1, jnp.int32)

        noff_f, _ = _p2
        cl = lax.bitwise_and(
            lax.shift_right_logical(lax.reduce_max(noff_f, (0,)), 7), LR - 1)
        pltpu.async_copy(upd_hbm.at[plist.at[cl]], rows, sem).wait()
        pltpu.async_copy(rows, mem_ref.at[rlist.at[cl]], sem).wait()

        pltpu.async_copy(frows, mem_ref.at[fidx], sem).wait()
        w0 = winp[pl.ds(0, 16)]
        val = lax.reduce_max(
            jnp.where(lane == 0, w0, jnp.full((16,), -2147483647, jnp.int32)),
            (0,))

        @pl.when(val >= 0)
        def _fix():
            fidx[...] = jnp.full((16,), val, jnp.int32)
            pltpu.async_copy(upd_hbm.at[fidx], frows, sem).wait()
            fidx[...] = jnp.full((16,), base, jnp.int32)
            pltpu.async_copy(frows, mem_ref.at[fidx], sem).wait()

    return scatter


def kernel(mem, val, Wz, Wr, Wq, idx):
    M, D = mem.shape
    B = idx.shape[0]
    idx = idx.astype(jnp.int32)

    h = _make_gather(M, B, D, 512)(mem, idx.reshape(B // 128, 128))

    BLK = 6912
    h_new = pl.pallas_call(
        _gru_block,
        grid=(B // BLK,),
        in_specs=[
            pl.BlockSpec((BLK, D), lambda i: (i, 0)),
            pl.BlockSpec((BLK, D), lambda i: (i, 0)),
            pl.BlockSpec((2 * D, D), lambda i: (0, 0)),
            pl.BlockSpec((2 * D, D), lambda i: (0, 0)),
            pl.BlockSpec((2 * D, D), lambda i: (0, 0)),
        ],
        out_specs=pl.BlockSpec((BLK, D), lambda i: (i, 0)),
        out_shape=jax.ShapeDtypeStruct((B, D), jnp.float32),
    )(h, val, Wz, Wr, Wq)

    mem_ref = jax.new_ref(mem)
    _make_scatter(M, B, D, 8192, 128)(mem_ref, idx, h_new)
    return mem_ref[...]


# speedup vs baseline: 1.3176x; 1.0006x over previous
import functools

import jax
import jax.numpy as jnp
from jax import lax
from jax.experimental import pallas as pl
from jax.experimental.pallas import tpu as pltpu
from jax.experimental.pallas import tpu_sc as plsc

_NC = 2
_NS = 16
_NW = _NC * _NS


def _wid():
    return lax.axis_index("s") * _NC + lax.axis_index("c")


def _gru_block(h_ref, v_ref, wz_ref, wr_ref, wq_ref, o_ref):
    h = h_ref[...]
    v = v_ref[...]
    Wz = wz_ref[...]
    Wr = wr_ref[...]
    Wq = wq_ref[...]
    D = h.shape[1]

    def mm(a, b):
        return jax.lax.dot_general(
            a, b, (((1,), (0,)), ((), ())), preferred_element_type=jnp.float32
        )

    z = jax.nn.sigmoid(mm(h, Wz[:D]) + mm(v, Wz[D:]))
    r = jax.nn.sigmoid(mm(h, Wr[:D]) + mm(v, Wr[D:]))
    q = jnp.tanh(mm(r * h, Wq[:D]) + mm(v, Wq[D:]))
    o_ref[...] = (1.0 - z) * h + z * q


@functools.cache
def _make_gather(M, B, D, W):
    CPW = B // _NW
    NWIN = CPW // W
    K = W // 128
    mesh = plsc.VectorSubcoreMesh(core_axis_name="c", subcore_axis_name="s")

    @functools.partial(
        pl.kernel,
        out_type=jax.ShapeDtypeStruct((B, D), jnp.float32),
        mesh=mesh,
        scratch_types=[
            pltpu.VMEM((K, 128), jnp.int32),
            pltpu.VMEM((W, D), jnp.float32),
            pltpu.SemaphoreType.DMA,
        ],
        compiler_params=pltpu.CompilerParams(use_tc_tiling_on_sc=False),
    )
    def gather(mem_hbm, idx_hbm, out_hbm, idx_v, rows_v, sem):
        w = _wid()
        rbase = w * (CPW // 128)

        @pl.loop(0, NWIN)
        def _win(g):
            r0 = rbase + g * K
            pltpu.sync_copy(idx_hbm.at[pl.ds(r0, K)], idx_v)
            cps = [
                pltpu.async_copy(
                    mem_hbm.at[idx_v.at[j]],
                    rows_v.at[pl.ds(j * 128, 128)],
                    sem,
                )
                for j in range(K)
            ]
            for cp in cps:
                cp.wait()
            pltpu.sync_copy(rows_v, out_hbm.at[pl.ds(r0 * 128, W)])

    return gather


@functools.cache
def _make_scatter(M, B, D, IW, CH):
    MW = M // _NW
    NWIN = B // IW
    NV = IW // 16
    LR = IW // CH
    mesh = plsc.VectorSubcoreMesh(core_axis_name="c", subcore_axis_name="s")

    @functools.partial(
        pl.kernel,
        out_type=(),
        mesh=mesh,
        scratch_types=[
            pltpu.VMEM((MW,), jnp.int32),
            pltpu.VMEM((IW,), jnp.int32),
            pltpu.VMEM((IW,), jnp.int32),
            pltpu.VMEM((LR, CH), jnp.int32),
            pltpu.VMEM((LR, CH), jnp.int32),
            pltpu.VMEM((CH, D), jnp.float32),
            pltpu.VMEM((16, D), jnp.float32),
            pltpu.VMEM((16,), jnp.int32),
            pltpu.SemaphoreType.DMA,
            pltpu.SemaphoreType.DMA,
            pltpu.SemaphoreType.DMA,
        ],
        compiler_params=pltpu.CompilerParams(
            use_tc_tiling_on_sc=False, needs_layout_passes=False),
    )
    def scatter(mem_ref, idx_hbm, upd_hbm,
                winp, idxv0, idxv1, plist, rlist, rows, frows, fidx, sem, sem0, sem1):
        w = _wid()
        base = w * MW
        lane = lax.iota(jnp.int32, 16)

        fidx[...] = jnp.full((16,), base, jnp.int32)
        pltpu.async_copy(mem_ref.at[fidx], frows, sem).wait()
        @pl.loop(0, MW // 16, unroll=16)
        def _init(i):
            winp[pl.ds(i * 16, 16)] = jnp.full((16,), -1, jnp.int32)

        def _p1_body(g, buf):
            @pl.loop(0, NV, unroll=16)
            def _v(j):
                v = buf[pl.ds(j * 16, 16)]
                a = v - base
                m = (a >= 0) & (a < MW)
                _, lastm = plsc.scan_count(v, m)
                posv = (g * IW + j * 16) + lane
                plsc.store_scatter(winp, [a], posv, mask=m & lastm)

        def _load(g, buf, s):
            gw = lax.rem(g, NWIN)
            pltpu.async_copy(idx_hbm.at[pl.ds(gw * IW, IW)], buf, s)

        def _drain(buf, s):
            pltpu.make_async_copy(idx_hbm.at[pl.ds(0, IW)], buf, s).wait()

        _load(0, idxv0, sem0)

        @pl.loop(0, NWIN // 2)
        def _p1(t):
            g0 = 2 * t
            _drain(idxv0, sem0)
            _load(g0 + 1, idxv1, sem1)
            _p1_body(g0, idxv0)
            _drain(idxv1, sem1)
            _load(g0 + 2, idxv0, sem0)
            _p1_body(g0 + 1, idxv1)

        _drain(idxv0, sem0)

        for r in range(LR):
            for c in range(CH // 16):
                plist[r, pl.ds(c * 16, 16)] = jnp.zeros((16,), jnp.int32)
                rlist[r, pl.ds(c * 16, 16)] = jnp.full((16,), base, jnp.int32)

        BK = 2048
        zero16 = jnp.zeros((16,), jnp.int32)

        @pl.loop(0, MW // BK, init_carry=(zero16, zero16))
        def _p2(bk, carry):
            noff, done = carry

            @pl.loop(0, BK // 16, unroll=8, init_carry=noff)
            def _vv(i, nf):
                off = bk * BK + i * 16
                wp = winp[pl.ds(off, 16)]
                k = wp >= 0
                tgt = nf + plsc.cumsum(k.astype(jnp.int32)) - 1
                tr = lax.bitwise_and(lax.shift_right_logical(tgt, 7), LR - 1)
                tc = lax.bitwise_and(tgt, 127)
                plsc.store_scatter(plist, [tr, tc], wp, mask=k)
                plsc.store_scatter(rlist, [tr, tc], (base + off) + lane,
                                   mask=k)
                return nf + plsc.all_reduce_population_count(k)

            d0 = lax.reduce_max(done, (0,))
            d1 = lax.reduce_max(lax.shift_right_logical(_vv, 7), (0,))

            @pl.loop(d0, d1)
            def _f(c):
                cr = lax.bitwise_and(c, LR - 1)
                pltpu.async_copy(upd_hbm.at[plist.at[cr]], rows, sem).wait()
                pltpu.async_copy(rows, mem_ref.at[rlist.at[cr]], sem).wait()

            return _vv, jnp.full((16,), d1, jnp.int32)

        noff_f, _ = _p2
        cl = lax.bitwise_and(
            lax.shift_right_logical(lax.reduce_max(noff_f, (0,)), 7), LR - 1)
        pltpu.async_copy(upd_hbm.at[plist.at[cl]], rows, sem).wait()
        pltpu.async_copy(rows, mem_ref.at[rlist.at[cl]], sem).wait()

        pltpu.async_copy(frows, mem_ref.at[fidx], sem).wait()
        w0 = winp[pl.ds(0, 16)]
        val = lax.reduce_max(
            jnp.where(lane == 0, w0, jnp.full((16,), -2147483647, jnp.int32)),
            (0,))

        @pl.when(val >= 0)
        def _fix():
            fidx[...] = jnp.full((16,), val, jnp.int32)
            pltpu.async_copy(upd_hbm.at[fidx], frows, sem).wait()
            fidx[...] = jnp.full((16,), base, jnp.int32)
            pltpu.async_copy(frows, mem_ref.at[fidx], sem).wait()

    return scatter


def kernel(mem, val, Wz, Wr, Wq, idx):
    M, D = mem.shape
    B = idx.shape[0]
    idx = idx.astype(jnp.int32)

    h = _make_gather(M, B, D, 512)(mem, idx.reshape(B // 128, 128))

    BLK = 6912
    h_new = pl.pallas_call(
        _gru_block,
        grid=(B // BLK,),
        in_specs=[
            pl.BlockSpec((BLK, D), lambda i: (i, 0)),
            pl.BlockSpec((BLK, D), lambda i: (i, 0)),
            pl.BlockSpec((2 * D, D), lambda i: (0, 0)),
            pl.BlockSpec((2 * D, D), lambda i: (0, 0)),
            pl.BlockSpec((2 * D, D), lambda i: (0, 0)),
        ],
        out_specs=pl.BlockSpec((BLK, D), lambda i: (i, 0)),
        out_shape=jax.ShapeDtypeStruct((B, D), jnp.float32),
    )(h, val, Wz, Wr, Wq)

    mem_ref = jax.new_ref(mem)
    _make_scatter(M, B, D, 8192, 128)(mem_ref, idx, h_new)
    return mem_ref[...]
